# baseline (speedup 1.0000x reference)
import functools

import jax
import jax.numpy as jnp
from jax import lax
from jax.experimental import pallas as pl
from jax.experimental.pallas import tpu as pltpu
from jax.experimental.pallas import tpu_sc as plsc

NC = 2
NS = 16
NW = NC * NS
LB = 128
K = 4


def _iota16():
    return lax.iota(jnp.int32, 16)


def _gather_body(table_hbm, x_hbm, out_hbm, idx_v, idx_t, gbuf, tbuf,
                 gsA, gsB, osA, osB, *, nb, hist, emb):
    wid = lax.axis_index("s") * NC + lax.axis_index("c")
    base = wid * nb
    nj = nb // LB
    units = nj * hist
    rounds = units // (2 * K)
    tpu_blk = 8 * LB
    usz = emb * LB

    pltpu.sync_copy(x_hbm.at[pl.ds(base, nb)], idx_v)

    @plsc.parallel_loop(0, hist, step=1, unroll=2)
    def build_idx_t(h):
        cols = jnp.full((16,), h, jnp.int32)
        for kb in range(nb // 16):
            rows = _iota16() + kb * 16
            v = plsc.load_gather(idx_v, [rows, cols])
            idx_t[pl.ds(h * nb + kb * 16, 16)] = v

    def unit_hj(u):
        return u // hist, u % hist

    def fire_gather(u, slot, sem):
        jj, h = unit_hj(u)
        off = h * nb + jj * LB
        pltpu.async_copy(table_hbm.at[idx_t.at[pl.ds(off, LB)]],
                         gbuf.at[slot], sem)

    def drain_gather(slot, sem):
        pltpu.make_async_copy(table_hbm.at[pl.ds(0, LB)], gbuf.at[slot],
                              sem).wait()

    def transpose_unit(slot):
        g = gbuf.at[slot]
        sb = slot * usz
        cbs = [(_iota16() + c * 16) * LB + sb for c in range(emb // 16)]

        @plsc.parallel_loop(0, LB, step=1, unroll=8)
        def tr(l):
            for c in range(emb // 16):
                v = g[l, pl.ds(c * 16, 16)]
                plsc.store_scatter(tbuf, [cbs[c] + l], v)

    def fire_out(u, slot, sem):
        jj, h = unit_hj(u)
        jg = wid * nj + jj
        for i in range(emb // 8):
            pltpu.async_copy(
                tbuf.at[pl.ds(slot * usz + i * tpu_blk, tpu_blk)],
                out_hbm.at[pl.ds(((h * (emb // 8) + i) * (NW * nj) + jg)
                                 * tpu_blk, tpu_blk)],
                sem)

    def drain_out(slot, sem):
        for i in range(emb // 8):
            pltpu.make_async_copy(
                out_hbm.at[pl.ds(i * tpu_blk, tpu_blk)],
                tbuf.at[pl.ds(slot * usz + i * tpu_blk, tpu_blk)],
                sem).wait()

    def round_body(t, *, first, last):
        uA = 2 * K * t
        uB = uA + K
        for b in range(K):
            drain_gather(b, gsA)
        if not first:
            for b in range(K):
                drain_out(b, osA)
        for b in range(K):
            transpose_unit(b)
        for b in range(K):
            fire_out(uA + b, b, osA)
        for b in range(K):
            drain_gather(K + b, gsB)
        if not first:
            for b in range(K):
                drain_out(K + b, osB)
        for b in range(K):
            transpose_unit(K + b)
        for b in range(K):
            fire_out(uB + b, K + b, osB)
        if not last:
            for b in range(K):
                fire_gather(uA + 2 * K + b, b, gsA)
            for b in range(K):
                fire_gather(uB + 2 * K + b, K + b, gsB)

    for b in range(K):
        fire_gather(b, b, gsA)
    for b in range(K):
        fire_gather(K + b, K + b, gsB)

    round_body(0, first=True, last=(rounds == 1))

    def mid(t, carry):
        round_body(t, first=False, last=False)
        return carry

    if rounds > 2:
        lax.fori_loop(1, rounds - 1, mid, 0)
    if rounds > 1:
        round_body(rounds - 1, first=False, last=True)

    for b in range(K):
        drain_out(b, osA)
    for b in range(K):
        drain_out(K + b, osB)


def kernel(x, table):
    bsz, hist = x.shape
    vocab, emb = table.shape
    assert bsz % (NW * LB) == 0 and emb % 16 == 0
    nb = bsz // NW
    nj = nb // LB
    assert (nj * hist) % (2 * K) == 0

    mesh = plsc.VectorSubcoreMesh(core_axis_name="c", subcore_axis_name="s")
    k = pl.kernel(
        functools.partial(_gather_body, nb=nb, hist=hist, emb=emb),
        out_type=jax.ShapeDtypeStruct((bsz * hist * emb,), jnp.float32),
        mesh=mesh,
        scratch_types=[
            pltpu.VMEM((nb, hist), jnp.int32),
            pltpu.VMEM((nb * hist,), jnp.int32),
            pltpu.VMEM((2 * K, LB, emb), jnp.float32),
            pltpu.VMEM((2 * K * LB * emb,), jnp.float32),
            pltpu.SemaphoreType.DMA,
            pltpu.SemaphoreType.DMA,
            pltpu.SemaphoreType.DMA,
            pltpu.SemaphoreType.DMA,
        ],
        compiler_params=pltpu.CompilerParams(use_tc_tiling_on_sc=False,
                                             needs_layout_passes=False),
    )
    flat = k(table, x.astype(jnp.int32))
    out5 = flat.reshape(hist, emb // 8, bsz // LB, 8, LB)
    return jnp.transpose(out5, (2, 4, 0, 1, 3)).reshape(bsz, hist, emb)


# speedup vs baseline: 1.0558x; 1.0558x over previous
import functools

import jax
import jax.numpy as jnp
from jax import lax
from jax.experimental import pallas as pl
from jax.experimental.pallas import tpu as pltpu
from jax.experimental.pallas import tpu_sc as plsc

NC = 2
NS = 16
NW = NC * NS
LB = 128
K = 4


def _iota16():
    return lax.iota(jnp.int32, 16)


def _gather_body(table_hbm, x_hbm, out_hbm, idx_v, idx_t, gbuf, tbuf,
                 gsA, gsB, osA, osB, *, nb, hist, emb):
    wid = lax.axis_index("s") * NC + lax.axis_index("c")
    base = wid * nb
    nj = nb // LB
    units = nj * hist
    rounds = units // (2 * K)
    tpu_blk = 8 * LB
    usz = emb * LB

    pltpu.sync_copy(x_hbm.at[pl.ds(base, nb)], idx_v)

    @plsc.parallel_loop(0, hist, step=1, unroll=2)
    def build_idx_t(h):
        cols = jnp.full((16,), h, jnp.int32)
        for kb in range(nb // 16):
            rows = _iota16() + kb * 16
            v = plsc.load_gather(idx_v, [rows, cols])
            idx_t[pl.ds(h * nb + kb * 16, 16)] = v

    def unit_hj(u):
        return u // hist, u % hist

    def fire_gather(u, slot, sem):
        jj, h = unit_hj(u)
        off = h * nb + jj * LB
        pltpu.async_copy(table_hbm.at[idx_t.at[pl.ds(off, LB)]],
                         gbuf.at[slot], sem)

    def drain_gather(slot, sem):
        pltpu.make_async_copy(table_hbm.at[pl.ds(0, LB)], gbuf.at[slot],
                              sem).wait()

    def transpose_unit(slot):
        g = gbuf.at[slot]
        sb = slot * usz
        cbs = [(_iota16() + c * 16) * LB + sb for c in range(emb // 16)]

        @plsc.parallel_loop(0, LB, step=1, unroll=8)
        def tr(l):
            for c in range(emb // 16):
                v = g[l, pl.ds(c * 16, 16)]
                plsc.store_scatter(tbuf, [cbs[c] + l], v)

    def fire_out(u, slot, sem):
        jj, h = unit_hj(u)
        jg = wid * nj + jj
        for i in range(emb // 8):
            pltpu.async_copy(
                tbuf.at[pl.ds(slot * usz + i * tpu_blk, tpu_blk)],
                out_hbm.at[pl.ds(((h * (emb // 8) + i) * (NW * nj) + jg)
                                 * tpu_blk, tpu_blk)],
                sem)

    def drain_out(slot, sem):
        for i in range(emb // 8):
            pltpu.make_async_copy(
                out_hbm.at[pl.ds(i * tpu_blk, tpu_blk)],
                tbuf.at[pl.ds(slot * usz + i * tpu_blk, tpu_blk)],
                sem).wait()

    def round_body(t, *, first, last):
        uA = 2 * K * t
        uB = uA + K
        for b in range(K):
            drain_gather(b, gsA)
        if not first:
            for b in range(K):
                drain_out(b, osA)
        for b in range(K):
            transpose_unit(b)
        if not last:
            for b in range(K):
                fire_gather(uA + 2 * K + b, b, gsA)
        for b in range(K):
            fire_out(uA + b, b, osA)
        for b in range(K):
            drain_gather(K + b, gsB)
        if not first:
            for b in range(K):
                drain_out(K + b, osB)
        for b in range(K):
            transpose_unit(K + b)
        if not last:
            for b in range(K):
                fire_gather(uB + 2 * K + b, K + b, gsB)
        for b in range(K):
            fire_out(uB + b, K + b, osB)

    for b in range(K):
        fire_gather(b, b, gsA)
    for b in range(K):
        fire_gather(K + b, K + b, gsB)

    round_body(0, first=True, last=(rounds == 1))

    def mid(t, carry):
        round_body(t, first=False, last=False)
        return carry

    if rounds > 2:
        lax.fori_loop(1, rounds - 1, mid, 0)
    if rounds > 1:
        round_body(rounds - 1, first=False, last=True)

    for b in range(K):
        drain_out(b, osA)
    for b in range(K):
        drain_out(K + b, osB)


def kernel(x, table):
    bsz, hist = x.shape
    vocab, emb = table.shape
    assert bsz % (NW * LB) == 0 and emb % 16 == 0
    nb = bsz // NW
    nj = nb // LB
    assert (nj * hist) % (2 * K) == 0

    mesh = plsc.VectorSubcoreMesh(core_axis_name="c", subcore_axis_name="s")
    k = pl.kernel(
        functools.partial(_gather_body, nb=nb, hist=hist, emb=emb),
        out_type=jax.ShapeDtypeStruct((bsz * hist * emb,), jnp.float32),
        mesh=mesh,
        scratch_types=[
            pltpu.VMEM((nb, hist), jnp.int32),
            pltpu.VMEM((nb * hist,), jnp.int32),
            pltpu.VMEM((2 * K, LB, emb), jnp.float32),
            pltpu.VMEM((2 * K * LB * emb,), jnp.float32),
            pltpu.SemaphoreType.DMA,
            pltpu.SemaphoreType.DMA,
            pltpu.SemaphoreType.DMA,
            pltpu.SemaphoreType.DMA,
        ],
        compiler_params=pltpu.CompilerParams(use_tc_tiling_on_sc=False,
                                             needs_layout_passes=False),
    )
    flat = k(table, x.astype(jnp.int32))
    out5 = flat.reshape(hist, emb // 8, bsz // LB, 8, LB)
    return jnp.transpose(out5, (2, 4, 0, 1, 3)).reshape(bsz, hist, emb)


# speedup vs baseline: 1.4462x; 1.3698x over previous
import functools

import jax
import jax.numpy as jnp
import numpy as np
from jax import lax
from jax.experimental import pallas as pl
from jax.experimental.pallas import tpu as pltpu
from jax.experimental.pallas import tpu_sc as plsc

NC = 2
NS = 16
NW = NC * NS
LB = 128
K = 4


def _iota16():
    return lax.iota(jnp.int32, 16)


def _gather_body(table_hbm, x_hbm, out_hbm, idx_v, idx_t, gbuf, tbuf, dtab,
                 gsA, gsB, osA, osB, *, nb, hist, emb):
    wid = lax.axis_index("s") * NC + lax.axis_index("c")
    base = wid * nb
    nj = nb // LB
    units = nj * hist
    rounds = units // (2 * K)
    tpu_blk = 8 * LB
    usz = emb * LB
    ncd = (emb // 16) * 16 * 16

    it = _iota16()
    for c in range(emb // 16):
        for d in range(16):
            colv = ((it + d) & 15) + c * 16
            dtab[pl.ds((c * 16 + d) * 16, 16)] = colv
            dtab[pl.ds(ncd + (c * 16 + d) * 16, 16)] = colv * LB + it

    pltpu.sync_copy(x_hbm.at[pl.ds(base, nb)], idx_v)

    @plsc.parallel_loop(0, hist, step=1, unroll=2)
    def build_idx_t(h):
        cols = jnp.full((16,), h, jnp.int32)
        for kb in range(nb // 16):
            rows = _iota16() + kb * 16
            v = plsc.load_gather(idx_v, [rows, cols])
            idx_t[pl.ds(h * nb + kb * 16, 16)] = v

    def unit_hj(u):
        return u // hist, u % hist

    def fire_gather(u, slot, sem):
        jj, h = unit_hj(u)
        off = h * nb + jj * LB
        pltpu.async_copy(table_hbm.at[idx_t.at[pl.ds(off, LB)]],
                         gbuf.at[slot], sem)

    def drain_gather(slot, sem):
        pltpu.make_async_copy(table_hbm.at[pl.ds(0, LB)], gbuf.at[slot],
                              sem).wait()

    def transpose_half(s0):
        def per_slot(b, carry):
            slot = s0 + b
            sv = jnp.full((16,), slot, jnp.int32)
            sb = slot * usz

            @plsc.parallel_loop(0, (LB // 16) * (emb // 16), step=1, unroll=2)
            def tr(q):
                l0 = q // (emb // 16)
                c = q % (emb // 16)
                rv = _iota16() + l0 * 16
                sc = sb + l0 * 16
                for d in range(16):
                    cd = (c * 16 + d) * 16
                    colv = dtab[pl.ds(cd, 16)]
                    dstv = dtab[pl.ds(ncd + cd, 16)]
                    v = plsc.load_gather(gbuf, [sv, rv, colv])
                    plsc.store_scatter(tbuf, [dstv + sc], v)
            return carry

        lax.fori_loop(0, K, per_slot, 0)

    def fire_out(u, slot, sem):
        jj, h = unit_hj(u)
        jg = wid * nj + jj
        for i in range(emb // 8):
            pltpu.async_copy(
                tbuf.at[pl.ds(slot * usz + i * tpu_blk, tpu_blk)],
                out_hbm.at[pl.ds(((h * (emb // 8) + i) * (NW * nj) + jg)
                                 * tpu_blk, tpu_blk)],
                sem)

    def drain_out(slot, sem):
        for i in range(emb // 8):
            pltpu.make_async_copy(
                out_hbm.at[pl.ds(i * tpu_blk, tpu_blk)],
                tbuf.at[pl.ds(slot * usz + i * tpu_blk, tpu_blk)],
                sem).wait()

    for b in range(K):
        fire_gather(b, b, gsA)
    for b in range(K):
        fire_gather(K + b, K + b, gsB)

    def round_body(t, carry):
        uA = 2 * K * t
        uB = uA + K
        for b in range(K):
            drain_gather(b, gsA)

        @pl.when(t > 0)
        def _():
            for b in range(K):
                drain_out(b, osA)

        transpose_half(0)

        @pl.when(t < rounds - 1)
        def _():
            for b in range(K):
                fire_gather(uA + 2 * K + b, b, gsA)

        for b in range(K):
            fire_out(uA + b, b, osA)
        for b in range(K):
            drain_gather(K + b, gsB)

        @pl.when(t > 0)
        def _():
            for b in range(K):
                drain_out(K + b, osB)

        transpose_half(K)

        @pl.when(t < rounds - 1)
        def _():
            for b in range(K):
                fire_gather(uB + 2 * K + b, K + b, gsB)

        for b in range(K):
            fire_out(uB + b, K + b, osB)
        return carry

    lax.fori_loop(0, rounds, round_body, 0)

    for b in range(K):
        drain_out(b, osA)
    for b in range(K):
        drain_out(K + b, osB)


def kernel(x, table):
    bsz, hist = x.shape
    vocab, emb = table.shape
    assert bsz % (NW * LB) == 0 and emb % 16 == 0
    nb = bsz // NW
    nj = nb // LB
    assert (nj * hist) % (2 * K) == 0

    mesh = plsc.VectorSubcoreMesh(core_axis_name="c", subcore_axis_name="s")
    k = pl.kernel(
        functools.partial(_gather_body, nb=nb, hist=hist, emb=emb),
        out_type=jax.ShapeDtypeStruct((bsz * hist * emb,), jnp.float32),
        mesh=mesh,
        scratch_types=[
            pltpu.VMEM((nb, hist), jnp.int32),
            pltpu.VMEM((nb * hist,), jnp.int32),
            pltpu.VMEM((2 * K, LB, emb), jnp.float32),
            pltpu.VMEM((2 * K * LB * emb,), jnp.float32),
            pltpu.VMEM((2 * (emb // 16) * 16 * 16,), jnp.int32),
            pltpu.SemaphoreType.DMA,
            pltpu.SemaphoreType.DMA,
            pltpu.SemaphoreType.DMA,
            pltpu.SemaphoreType.DMA,
        ],
        compiler_params=pltpu.CompilerParams(use_tc_tiling_on_sc=False,
                                             needs_layout_passes=False),
    )
    flat = k(table, x.astype(jnp.int32))
    out5 = flat.reshape(hist, emb // 8, bsz // LB, 8, LB)
    return jnp.transpose(out5, (2, 4, 0, 1, 3)).reshape(bsz, hist, emb)


# speedup vs baseline: 1.4479x; 1.0012x over previous
import functools

import jax
import jax.numpy as jnp
from jax import lax
from jax.experimental import pallas as pl
from jax.experimental.pallas import tpu as pltpu
from jax.experimental.pallas import tpu_sc as plsc

NC = 2
NS = 16
NW = NC * NS
LB = 128
K = 4


def _iota16():
    return lax.iota(jnp.int32, 16)


def _gather_body(table_hbm, x_hbm, out_hbm, idx_v, idx_t, gbuf, tbuf, dtab,
                 gsA, gsB, osA, osB, *, nb, hist, emb):
    wid = lax.axis_index("s") * NC + lax.axis_index("c")
    base = wid * nb
    nj = nb // LB
    units = nj * hist
    rounds = units // (2 * K)
    tpu_blk = 8 * LB
    usz = emb * LB
    ncd = (emb // 16) * 16 * 16

    it = _iota16()
    for c in range(emb // 16):
        for d in range(16):
            colv = ((it + d) & 15) + c * 16
            dtab[pl.ds((c * 16 + d) * 16, 16)] = colv
            dtab[pl.ds(ncd + (c * 16 + d) * 16, 16)] = colv * LB + it

    pltpu.sync_copy(x_hbm.at[pl.ds(base, nb)], idx_v)

    @plsc.parallel_loop(0, hist, step=1, unroll=2)
    def build_idx_t(h):
        cols = jnp.full((16,), h, jnp.int32)
        for kb in range(nb // 16):
            rows = _iota16() + kb * 16
            v = plsc.load_gather(idx_v, [rows, cols])
            idx_t[pl.ds(h * nb + kb * 16, 16)] = v

    def unit_hj(u):
        return u // hist, u % hist

    def fire_gather(u, slot, sem):
        jj, h = unit_hj(u)
        off = h * nb + jj * LB
        pltpu.async_copy(table_hbm.at[idx_t.at[pl.ds(off, LB)]],
                         gbuf.at[slot], sem)

    def drain_gather(slot, sem):
        pltpu.make_async_copy(table_hbm.at[pl.ds(0, LB)], gbuf.at[slot],
                              sem).wait()

    def transpose_half(s0):
        def per_slot(b, carry):
            slot = s0 + b
            sv = jnp.full((16,), slot, jnp.int32)
            sb = slot * usz

            @plsc.parallel_loop(0, (LB // 16) * (emb // 16), step=1, unroll=2)
            def tr(q):
                l0 = q // (emb // 16)
                c = q % (emb // 16)
                rv = _iota16() + l0 * 16
                sc = sb + l0 * 16
                for d in range(16):
                    cd = (c * 16 + d) * 16
                    colv = dtab[pl.ds(cd, 16)]
                    dstv = dtab[pl.ds(ncd + cd, 16)]
                    v = plsc.load_gather(gbuf, [sv, rv, colv])
                    plsc.store_scatter(tbuf, [dstv + sc], v)
            return carry

        lax.fori_loop(0, K, per_slot, 0)

    def fire_out(u, slot, sem):
        jj, h = unit_hj(u)
        jg = wid * nj + jj
        for i in range(emb // 8):
            pltpu.async_copy(
                tbuf.at[pl.ds(slot * usz + i * tpu_blk, tpu_blk)],
                out_hbm.at[pl.ds(((h * (emb // 8) + i) * (NW * nj) + jg)
                                 * tpu_blk, tpu_blk)],
                sem)

    def drain_out(slot, sem):
        for i in range(emb // 8):
            pltpu.make_async_copy(
                out_hbm.at[pl.ds(i * tpu_blk, tpu_blk)],
                tbuf.at[pl.ds(slot * usz + i * tpu_blk, tpu_blk)],
                sem).wait()

    for b in range(K):
        fire_gather(b, b, gsA)
    for b in range(K):
        fire_gather(K + b, K + b, gsB)

    def round_body(t, carry):
        uA = 2 * K * t
        uB = uA + K
        for b in range(K):
            drain_gather(b, gsA)

        @pl.when(t > 0)
        def _():
            for b in range(K):
                drain_out(b, osA)

        transpose_half(0)

        @pl.when(t < rounds - 1)
        def _():
            for b in range(K):
                fire_gather(uA + 2 * K + b, b, gsA)

        for b in range(K):
            fire_out(uA + b, b, osA)
        for b in range(K):
            drain_gather(K + b, gsB)

        @pl.when(t > 0)
        def _():
            for b in range(K):
                drain_out(K + b, osB)

        transpose_half(K)

        @pl.when(t < rounds - 1)
        def _():
            for b in range(K):
                fire_gather(uB + 2 * K + b, K + b, gsB)

        for b in range(K):
            fire_out(uB + b, K + b, osB)
        return carry

    lax.fori_loop(0, rounds, round_body, 0)

    for b in range(K):
        drain_out(b, osA)
    for b in range(K):
        drain_out(K + b, osB)


def kernel(x, table):
    bsz, hist = x.shape
    vocab, emb = table.shape
    assert bsz % (NW * LB) == 0 and emb % 16 == 0
    nb = bsz // NW
    nj = nb // LB
    assert (nj * hist) % (2 * K) == 0

    mesh = plsc.VectorSubcoreMesh(core_axis_name="c", subcore_axis_name="s")
    k = pl.kernel(
        functools.partial(_gather_body, nb=nb, hist=hist, emb=emb),
        out_type=jax.ShapeDtypeStruct((bsz * hist * emb,), jnp.float32),
        mesh=mesh,
        scratch_types=[
            pltpu.VMEM((nb, hist), jnp.int32),
            pltpu.VMEM((nb * hist,), jnp.int32),
            pltpu.VMEM((2 * K, LB, emb), jnp.float32),
            pltpu.VMEM((2 * K * LB * emb,), jnp.float32),
            pltpu.VMEM((2 * (emb // 16) * 16 * 16,), jnp.int32),
            pltpu.SemaphoreType.DMA,
            pltpu.SemaphoreType.DMA,
            pltpu.SemaphoreType.DMA,
            pltpu.SemaphoreType.DMA,
        ],
        compiler_params=pltpu.CompilerParams(use_tc_tiling_on_sc=False,
                                             needs_layout_passes=False),
    )
    flat = k(table, x.astype(jnp.int32))
    out5 = flat.reshape(hist, emb // 8, bsz // LB, 8, LB)
    return jnp.transpose(out5, (2, 4, 0, 1, 3)).reshape(bsz, hist, emb)
